# baseline (speedup 1.0000x reference)
import functools

import jax
import jax.numpy as jnp
from jax import lax
from jax.experimental import pallas as pl
from jax.experimental.pallas import tpu as pltpu
from jax.experimental.pallas import tpu_sc as plsc

_B = 16384
_N = 65
_H = 64
_TOK = _B * _N
_NC = 2
_NS = 16
_NW = _NC * _NS
_TPW = _TOK // _NW
_C = 320
_NCHUNK = _TPW // _C
_TBL = 4 * 256 * _H


def _prep_body(b0, b1, b2, b3, w, bias, role, t_out, role_out):
    wv = w[...]
    t_out[0:256, :] = jnp.dot(b0[...], wv[0:32, :],
                              preferred_element_type=jnp.float32)
    t_out[256:512, :] = jnp.dot(b1[...], wv[32:64, :],
                                preferred_element_type=jnp.float32)
    t_out[512:768, :] = jnp.dot(b2[...], wv[64:96, :],
                                preferred_element_type=jnp.float32)
    t_out[768:1024, :] = jnp.dot(b3[...], wv[96:128, :],
                                 preferred_element_type=jnp.float32)
    role_out[...] = role[...] + bias[...]


_prep = pl.pallas_call(
    _prep_body,
    out_shape=(
        jax.ShapeDtypeStruct((1024, _H), jnp.float32),
        jax.ShapeDtypeStruct((_N, _H), jnp.float32),
    ),
)


def _rsqrt16(v):
    i = plsc.bitcast(v, jnp.int32)
    y = plsc.bitcast(jnp.int32(0x5F3759DF) - (i >> 1), jnp.float32)
    half = v * 0.5
    for _ in range(2):
        y = y * (1.5 - half * y * y)
    return y


def _sc_body(sv_hbm, t_hbm, role_hbm, g_hbm, be_hbm, out_hbm,
             t_v, role_v, g_v, be_v, sv0, sv1, ob0, ob1,
             si0, si1, so0, so1):
    wid = lax.axis_index("s") * _NC + lax.axis_index("c")
    base = wid * _TPW

    pltpu.sync_copy(t_hbm, t_v)
    pltpu.sync_copy(role_hbm, role_v)
    pltpu.sync_copy(g_hbm, g_v)
    pltpu.sync_copy(be_hbm, be_v)

    g = [g_v[pl.ds(16 * j, 16)] for j in range(4)]
    be = [be_v[pl.ds(16 * j, 16)] for j in range(4)]
    lanes = jax.lax.broadcasted_iota(jnp.int32, (16,), 0)
    perms = [lanes ^ m for m in (8, 4, 2, 1)]

    def xsum(x):
        for p in perms:
            x = x + x.at[p].get(mode="promise_in_bounds")
        return x

    def in_copy(c, buf, sem):
        return pltpu.make_async_copy(
            sv_hbm.at[pl.ds(base + c * _C, _C)], buf.at[pl.ds(0, _C)], sem)

    def out_copy(c, buf, sem):
        return pltpu.make_async_copy(
            buf, out_hbm.at[pl.ds((base + c * _C) * _H, _C * _H)], sem)

    def inner(sv_v, out_v, n0):
        @plsc.parallel_loop(0, _C // 16, unroll=1)
        def group(gi):
            vv = sv_v[pl.ds(gi * 16, 16)]
            w0 = (vv & 255) * _H
            w1 = ((vv >> 8) & 255) * _H + 256 * _H
            w2 = ((vv >> 16) & 255) * _H + 512 * _H
            w3 = ((vv >> 24) & 255) * _H + 768 * _H
            nv = ((n0 + gi * 16 + lanes) % _N) * _H
            for kb in range(0, 16, 16):
                ys = []
                for k in range(kb, kb + 16):
                    o0 = w0[k]
                    o1 = w1[k]
                    o2 = w2[k]
                    o3 = w3[k]
                    ro = nv[k]
                    x = []
                    for j in range(4):
                        d = 16 * j
                        xj = ((t_v[pl.ds(o0 + d, 16)]
                               + t_v[pl.ds(o1 + d, 16)])
                              + (t_v[pl.ds(o2 + d, 16)]
                                 + t_v[pl.ds(o3 + d, 16)])
                              + role_v[pl.ds(ro + d, 16)])
                        x.append(xj)
                    s = (x[0] + x[1]) + (x[2] + x[3])
                    q = (x[0] * x[0] + x[1] * x[1]) \
                        + (x[2] * x[2] + x[3] * x[3])
                    mean = xsum(s) * (1.0 / _H)
                    var = xsum(q) * (1.0 / _H) - mean * mean + 1e-5
                    rstd = _rsqrt16(var)
                    ys.append([(x[j] - mean) * rstd * g[j] + be[j]
                               for j in range(4)])
                for k in range(kb, kb + 16):
                    ob = (gi * 16 + k) * _H
                    for j in range(4):
                        out_v[pl.ds(ob + 16 * j, 16)] = ys[k - kb][j]

        n0 = n0 + (_C % _N)
        return jnp.where(n0 >= _N, n0 - _N, n0)

    in_copy(0, sv0, si0).start()
    in_copy(1, sv1, si1).start()

    def cpair(cc, n0):
        for par, sv_v, out_v, si, so in ((0, sv0, ob0, si0, so0),
                                         (1, sv1, ob1, si1, so1)):
            c = cc * 2 + par
            in_copy(c, sv_v, si).wait()

            @pl.when(c >= 2)
            def _():
                out_copy(c - 2, out_v, so).wait()

            n0 = inner(sv_v, out_v, n0)
            out_copy(c, out_v, so).start()

            @pl.when(c + 2 < _NCHUNK)
            def _():
                in_copy(c + 2, sv_v, si).start()
        return n0

    lax.fori_loop(0, _NCHUNK // 2, cpair, 0)
    out_copy(_NCHUNK - 2, ob0, so0).wait()
    out_copy(_NCHUNK - 1, ob1, so1).wait()


_sc_call = pl.kernel(
    _sc_body,
    out_type=jax.ShapeDtypeStruct((_TOK * _H,), jnp.float32),
    mesh=plsc.VectorSubcoreMesh(core_axis_name="c", subcore_axis_name="s"),
    compiler_params=pltpu.CompilerParams(needs_layout_passes=False),
    scratch_types=[
        pltpu.VMEM((_TBL,), jnp.float32),
        pltpu.VMEM((_N * _H,), jnp.float32),
        pltpu.VMEM((_H,), jnp.float32),
        pltpu.VMEM((_H,), jnp.float32),
        pltpu.VMEM((_C + 16,), jnp.int32),
        pltpu.VMEM((_C + 16,), jnp.int32),
        pltpu.VMEM((_C * _H,), jnp.float32),
        pltpu.VMEM((_C * _H,), jnp.float32),
        pltpu.SemaphoreType.DMA,
        pltpu.SemaphoreType.DMA,
        pltpu.SemaphoreType.DMA,
        pltpu.SemaphoreType.DMA,
    ],
)


@jax.jit
def kernel(state_vals, role_table, byte0, byte1, byte2, byte3, W, b,
           gamma, beta):
    t_tbl, role_b = _prep(byte0, byte1, byte2, byte3, W,
                          b.reshape(1, _H), role_table)
    y = _sc_call(state_vals.reshape(_TOK), t_tbl.reshape(_TBL),
                 role_b.reshape(_N * _H), gamma, beta)
    return y.reshape(_B, _N, _H)


# speedup vs baseline: 1.0106x; 1.0106x over previous
import functools

import jax
import jax.numpy as jnp
from jax import lax
from jax.experimental import pallas as pl
from jax.experimental.pallas import tpu as pltpu
from jax.experimental.pallas import tpu_sc as plsc

_B = 16384
_N = 65
_H = 64
_TOK = _B * _N
_NC = 2
_NS = 16
_NW = _NC * _NS
_TPW = _TOK // _NW
_C = 320
_NCHUNK = _TPW // _C
_TBL = 4 * 256 * _H


def _prep_body(b0, b1, b2, b3, w, bias, role, t_out, role_out):
    wv = w[...]
    t_out[0:256, :] = jnp.dot(b0[...], wv[0:32, :],
                              preferred_element_type=jnp.float32)
    t_out[256:512, :] = jnp.dot(b1[...], wv[32:64, :],
                                preferred_element_type=jnp.float32)
    t_out[512:768, :] = jnp.dot(b2[...], wv[64:96, :],
                                preferred_element_type=jnp.float32)
    t_out[768:1024, :] = jnp.dot(b3[...], wv[96:128, :],
                                 preferred_element_type=jnp.float32)
    role_out[...] = role[...] + bias[...]


_prep = pl.pallas_call(
    _prep_body,
    out_shape=(
        jax.ShapeDtypeStruct((1024, _H), jnp.float32),
        jax.ShapeDtypeStruct((_N, _H), jnp.float32),
    ),
)


def _rsqrt16(v):
    i = plsc.bitcast(v, jnp.int32)
    y = plsc.bitcast(jnp.int32(0x5F3759DF) - (i >> 1), jnp.float32)
    half = v * 0.5
    for _ in range(2):
        y = y * (1.5 - half * y * y)
    return y


def _sc_body(sv_hbm, t_hbm, role_hbm, g_hbm, be_hbm, out_hbm,
             t_v, role_v, g_v, be_v, sv0, sv1, ob0, ob1,
             si0, si1, so0, so1):
    wid = lax.axis_index("s") * _NC + lax.axis_index("c")
    base = wid * _TPW

    pltpu.sync_copy(t_hbm, t_v)
    pltpu.sync_copy(role_hbm, role_v)
    pltpu.sync_copy(g_hbm, g_v)
    pltpu.sync_copy(be_hbm, be_v)

    g = [g_v[pl.ds(16 * j, 16)] for j in range(4)]
    be = [be_v[pl.ds(16 * j, 16)] for j in range(4)]
    lanes = jax.lax.broadcasted_iota(jnp.int32, (16,), 0)
    perms = [lanes ^ m for m in (8, 4, 2, 1)]

    def xsum(x):
        for p in perms:
            x = x + x.at[p].get(mode="promise_in_bounds")
        return x

    def in_copy(c, buf, sem):
        return pltpu.make_async_copy(
            sv_hbm.at[pl.ds(base + c * _C, _C)], buf.at[pl.ds(0, _C)], sem)

    def out_copy(c, buf, sem):
        return pltpu.make_async_copy(
            buf, out_hbm.at[pl.ds((base + c * _C) * _H, _C * _H)], sem)

    def inner(sv_v, out_v, n0):
        @plsc.parallel_loop(0, _C // 16, unroll=1)
        def group(gi):
            vv = sv_v[pl.ds(gi * 16, 16)]
            w0 = (vv & 255) * _H
            w1 = ((vv >> 8) & 255) * _H + 256 * _H
            w2 = ((vv >> 16) & 255) * _H + 512 * _H
            w3 = ((vv >> 24) & 255) * _H + 768 * _H
            nv = ((n0 + gi * 16 + lanes) % _N) * _H
            for kb in range(0, 16, 4):
                ys = []
                for k in range(kb, kb + 4):
                    o0 = w0[k]
                    o1 = w1[k]
                    o2 = w2[k]
                    o3 = w3[k]
                    ro = nv[k]
                    x = []
                    for j in range(4):
                        d = 16 * j
                        xj = ((t_v[pl.ds(o0 + d, 16)]
                               + t_v[pl.ds(o1 + d, 16)])
                              + (t_v[pl.ds(o2 + d, 16)]
                                 + t_v[pl.ds(o3 + d, 16)])
                              + role_v[pl.ds(ro + d, 16)])
                        x.append(xj)
                    s = (x[0] + x[1]) + (x[2] + x[3])
                    q = (x[0] * x[0] + x[1] * x[1]) \
                        + (x[2] * x[2] + x[3] * x[3])
                    mean = xsum(s) * (1.0 / _H)
                    var = xsum(q) * (1.0 / _H) - mean * mean + 1e-5
                    rstd = _rsqrt16(var)
                    ys.append([(x[j] - mean) * rstd * g[j] + be[j]
                               for j in range(4)])
                for k in range(kb, kb + 4):
                    ob = (gi * 16 + k) * _H
                    for j in range(4):
                        out_v[pl.ds(ob + 16 * j, 16)] = ys[k - kb][j]

        n0 = n0 + (_C % _N)
        return jnp.where(n0 >= _N, n0 - _N, n0)

    in_copy(0, sv0, si0).start()
    in_copy(1, sv1, si1).start()

    def cpair(cc, n0):
        for par, sv_v, out_v, si, so in ((0, sv0, ob0, si0, so0),
                                         (1, sv1, ob1, si1, so1)):
            c = cc * 2 + par
            in_copy(c, sv_v, si).wait()

            @pl.when(c >= 2)
            def _():
                out_copy(c - 2, out_v, so).wait()

            n0 = inner(sv_v, out_v, n0)
            out_copy(c, out_v, so).start()

            @pl.when(c + 2 < _NCHUNK)
            def _():
                in_copy(c + 2, sv_v, si).start()
        return n0

    lax.fori_loop(0, _NCHUNK // 2, cpair, 0)
    out_copy(_NCHUNK - 2, ob0, so0).wait()
    out_copy(_NCHUNK - 1, ob1, so1).wait()


_sc_call = pl.kernel(
    _sc_body,
    out_type=jax.ShapeDtypeStruct((_TOK * _H,), jnp.float32),
    mesh=plsc.VectorSubcoreMesh(core_axis_name="c", subcore_axis_name="s"),
    compiler_params=pltpu.CompilerParams(needs_layout_passes=False),
    scratch_types=[
        pltpu.VMEM((_TBL,), jnp.float32),
        pltpu.VMEM((_N * _H,), jnp.float32),
        pltpu.VMEM((_H,), jnp.float32),
        pltpu.VMEM((_H,), jnp.float32),
        pltpu.VMEM((_C + 16,), jnp.int32),
        pltpu.VMEM((_C + 16,), jnp.int32),
        pltpu.VMEM((_C * _H,), jnp.float32),
        pltpu.VMEM((_C * _H,), jnp.float32),
        pltpu.SemaphoreType.DMA,
        pltpu.SemaphoreType.DMA,
        pltpu.SemaphoreType.DMA,
        pltpu.SemaphoreType.DMA,
    ],
)


@jax.jit
def kernel(state_vals, role_table, byte0, byte1, byte2, byte3, W, b,
           gamma, beta):
    t_tbl, role_b = _prep(byte0, byte1, byte2, byte3, W,
                          b.reshape(1, _H), role_table)
    y = _sc_call(state_vals.reshape(_TOK), t_tbl.reshape(_TBL),
                 role_b.reshape(_N * _H), gamma, beta)
    return y.reshape(_B, _N, _H)


# speedup vs baseline: 1.0555x; 1.0444x over previous
import functools

import jax
import jax.numpy as jnp
from jax import lax
from jax.experimental import pallas as pl
from jax.experimental.pallas import tpu as pltpu
from jax.experimental.pallas import tpu_sc as plsc

_B = 16384
_N = 65
_H = 64
_TOK = _B * _N
_NC = 2
_NS = 16
_NW = _NC * _NS
_TPW = _TOK // _NW
_C = 320
_NCHUNK = _TPW // _C
_TBL = 4 * 256 * _H


def _prep_body(b0, b1, b2, b3, w, bias, role, t_out, role_out):
    wv = w[...]
    t_out[0:256, :] = jnp.dot(b0[...], wv[0:32, :],
                              preferred_element_type=jnp.float32)
    t_out[256:512, :] = jnp.dot(b1[...], wv[32:64, :],
                                preferred_element_type=jnp.float32)
    t_out[512:768, :] = jnp.dot(b2[...], wv[64:96, :],
                                preferred_element_type=jnp.float32)
    t_out[768:1024, :] = jnp.dot(b3[...], wv[96:128, :],
                                 preferred_element_type=jnp.float32)
    role_out[...] = role[...] + bias[...]


_prep = pl.pallas_call(
    _prep_body,
    out_shape=(
        jax.ShapeDtypeStruct((1024, _H), jnp.float32),
        jax.ShapeDtypeStruct((_N, _H), jnp.float32),
    ),
)


def _rsqrt16(v):
    i = plsc.bitcast(v, jnp.int32)
    y = plsc.bitcast(jnp.int32(0x5F3759DF) - (i >> 1), jnp.float32)
    half = v * 0.5
    for _ in range(2):
        y = y * (1.5 - half * y * y)
    return y


def _sc_body(sv_hbm, t_hbm, role_hbm, g_hbm, be_hbm, out_hbm,
             t_v, role_v, g_v, be_v, sv0, sv1, ob0, ob1,
             si0, si1, so0, so1):
    wid = lax.axis_index("s") * _NC + lax.axis_index("c")
    base = wid * _TPW

    pltpu.sync_copy(t_hbm, t_v)
    pltpu.sync_copy(role_hbm, role_v)
    pltpu.sync_copy(g_hbm, g_v)
    pltpu.sync_copy(be_hbm, be_v)

    g = [g_v[pl.ds(16 * j, 16)] for j in range(4)]
    be = [be_v[pl.ds(16 * j, 16)] for j in range(4)]
    lanes = jax.lax.broadcasted_iota(jnp.int32, (16,), 0)
    perms = [lanes ^ m for m in (8, 4, 2, 1)]

    def xsum(x):
        for p in perms:
            x = x + x.at[p].get(mode="promise_in_bounds")
        return x

    def in_copy(c, buf, sem):
        return pltpu.make_async_copy(
            sv_hbm.at[pl.ds(base + c * _C, _C)], buf.at[pl.ds(0, _C)], sem)

    def out_copy(c, buf, sem):
        return pltpu.make_async_copy(
            buf, out_hbm.at[pl.ds((base + c * _C) * _H, _C * _H)], sem)

    def inner(sv_v, out_v, n0):
        @plsc.parallel_loop(0, _C // 16, unroll=1)
        def group(gi):
            vv = sv_v[pl.ds(gi * 16, 16)]
            w0 = (vv & 255) * _H
            w1 = ((vv >> 8) & 255) * _H + 256 * _H
            w2 = ((vv >> 16) & 255) * _H + 512 * _H
            w3 = ((vv >> 24) & 255) * _H + 768 * _H
            nv = ((n0 + gi * 16 + lanes) % _N) * _H
            for kb in range(0, 16, 8):
                ys = []
                for k in range(kb, kb + 8):
                    o0 = w0[k]
                    o1 = w1[k]
                    o2 = w2[k]
                    o3 = w3[k]
                    ro = nv[k]
                    x = []
                    for j in range(4):
                        d = 16 * j
                        xj = ((t_v[pl.ds(o0 + d, 16)]
                               + t_v[pl.ds(o1 + d, 16)])
                              + (t_v[pl.ds(o2 + d, 16)]
                                 + t_v[pl.ds(o3 + d, 16)])
                              + role_v[pl.ds(ro + d, 16)])
                        x.append(xj)
                    s = (x[0] + x[1]) + (x[2] + x[3])
                    q = (x[0] * x[0] + x[1] * x[1]) \
                        + (x[2] * x[2] + x[3] * x[3])
                    mean = xsum(s) * (1.0 / _H)
                    var = xsum(q) * (1.0 / _H) - mean * mean + 1e-5
                    rstd = _rsqrt16(var)
                    ys.append([(x[j] - mean) * rstd * g[j] + be[j]
                               for j in range(4)])
                for k in range(kb, kb + 8):
                    ob = (gi * 16 + k) * _H
                    for j in range(4):
                        out_v[pl.ds(ob + 16 * j, 16)] = ys[k - kb][j]

        n0 = n0 + (_C % _N)
        return jnp.where(n0 >= _N, n0 - _N, n0)

    in_copy(0, sv0, si0).start()
    in_copy(1, sv1, si1).start()

    def cpair(cc, n0):
        for par, sv_v, out_v, si, so in ((0, sv0, ob0, si0, so0),
                                         (1, sv1, ob1, si1, so1)):
            c = cc * 2 + par
            in_copy(c, sv_v, si).wait()

            @pl.when(c >= 2)
            def _():
                out_copy(c - 2, out_v, so).wait()

            n0 = inner(sv_v, out_v, n0)
            out_copy(c, out_v, so).start()

            @pl.when(c + 2 < _NCHUNK)
            def _():
                in_copy(c + 2, sv_v, si).start()
        return n0

    lax.fori_loop(0, _NCHUNK // 2, cpair, 0)
    out_copy(_NCHUNK - 2, ob0, so0).wait()
    out_copy(_NCHUNK - 1, ob1, so1).wait()


_sc_call = pl.kernel(
    _sc_body,
    out_type=jax.ShapeDtypeStruct((_TOK * _H,), jnp.float32),
    mesh=plsc.VectorSubcoreMesh(core_axis_name="c", subcore_axis_name="s"),
    compiler_params=pltpu.CompilerParams(needs_layout_passes=False),
    scratch_types=[
        pltpu.VMEM((_TBL,), jnp.float32),
        pltpu.VMEM((_N * _H,), jnp.float32),
        pltpu.VMEM((_H,), jnp.float32),
        pltpu.VMEM((_H,), jnp.float32),
        pltpu.VMEM((_C + 16,), jnp.int32),
        pltpu.VMEM((_C + 16,), jnp.int32),
        pltpu.VMEM((_C * _H,), jnp.float32),
        pltpu.VMEM((_C * _H,), jnp.float32),
        pltpu.SemaphoreType.DMA,
        pltpu.SemaphoreType.DMA,
        pltpu.SemaphoreType.DMA,
        pltpu.SemaphoreType.DMA,
    ],
)


@jax.jit
def kernel(state_vals, role_table, byte0, byte1, byte2, byte3, W, b,
           gamma, beta):
    t_tbl, role_b = _prep(byte0, byte1, byte2, byte3, W,
                          b.reshape(1, _H), role_table)
    y = _sc_call(state_vals.reshape(_TOK), t_tbl.reshape(_TBL),
                 role_b.reshape(_N * _H), gamma, beta)
    return y.reshape(_B, _N, _H)


# speedup vs baseline: 1.0834x; 1.0265x over previous
import functools

import jax
import jax.numpy as jnp
from jax import lax
from jax.experimental import pallas as pl
from jax.experimental.pallas import tpu as pltpu
from jax.experimental.pallas import tpu_sc as plsc

_B = 16384
_N = 65
_H = 64
_TOK = _B * _N
_NC = 2
_NS = 16
_NW = _NC * _NS
_TPW = _TOK // _NW
_C = 320
_NCHUNK = _TPW // _C
_TBL = 4 * 256 * _H


def _prep_body(b0, b1, b2, b3, w, bias, role, t_out, role_out):
    wv = w[...]
    t_out[0:256, :] = jnp.dot(b0[...], wv[0:32, :],
                              preferred_element_type=jnp.float32)
    t_out[256:512, :] = jnp.dot(b1[...], wv[32:64, :],
                                preferred_element_type=jnp.float32)
    t_out[512:768, :] = jnp.dot(b2[...], wv[64:96, :],
                                preferred_element_type=jnp.float32)
    t_out[768:1024, :] = jnp.dot(b3[...], wv[96:128, :],
                                 preferred_element_type=jnp.float32)
    role_out[...] = role[...] + bias[...]


_prep = pl.pallas_call(
    _prep_body,
    out_shape=(
        jax.ShapeDtypeStruct((1024, _H), jnp.float32),
        jax.ShapeDtypeStruct((_N, _H), jnp.float32),
    ),
)


def _rsqrt16(v):
    i = plsc.bitcast(v, jnp.int32)
    y = plsc.bitcast(jnp.int32(0x5F3759DF) - (i >> 1), jnp.float32)
    half = v * 0.5
    for _ in range(1):
        y = y * (1.5 - half * y * y)
    return y


def _sc_body(sv_hbm, t_hbm, role_hbm, g_hbm, be_hbm, out_hbm,
             t_v, role_v, g_v, be_v, sv0, sv1, ob0, ob1,
             si0, si1, so0, so1):
    wid = lax.axis_index("s") * _NC + lax.axis_index("c")
    base = wid * _TPW

    pltpu.sync_copy(t_hbm, t_v)
    pltpu.sync_copy(role_hbm, role_v)
    pltpu.sync_copy(g_hbm, g_v)
    pltpu.sync_copy(be_hbm, be_v)

    g = [g_v[pl.ds(16 * j, 16)] for j in range(4)]
    be = [be_v[pl.ds(16 * j, 16)] for j in range(4)]
    lanes = jax.lax.broadcasted_iota(jnp.int32, (16,), 0)
    perms = [lanes ^ m for m in (8, 4, 2, 1)]

    def xsum(x):
        for p in perms:
            x = x + x.at[p].get(mode="promise_in_bounds")
        return x

    def in_copy(c, buf, sem):
        return pltpu.make_async_copy(
            sv_hbm.at[pl.ds(base + c * _C, _C)], buf.at[pl.ds(0, _C)], sem)

    def out_copy(c, buf, sem):
        return pltpu.make_async_copy(
            buf, out_hbm.at[pl.ds((base + c * _C) * _H, _C * _H)], sem)

    def inner(sv_v, out_v, n0):
        @plsc.parallel_loop(0, _C // 16, unroll=1)
        def group(gi):
            vv = sv_v[pl.ds(gi * 16, 16)]
            w0 = (vv & 255) * _H
            w1 = ((vv >> 8) & 255) * _H + 256 * _H
            w2 = ((vv >> 16) & 255) * _H + 512 * _H
            w3 = ((vv >> 24) & 255) * _H + 768 * _H
            nv = ((n0 + gi * 16 + lanes) % _N) * _H
            for kb in range(0, 16, 8):
                ys = []
                for k in range(kb, kb + 8):
                    o0 = w0[k]
                    o1 = w1[k]
                    o2 = w2[k]
                    o3 = w3[k]
                    ro = nv[k]
                    x = []
                    for j in range(4):
                        d = 16 * j
                        xj = ((t_v[pl.ds(o0 + d, 16)]
                               + t_v[pl.ds(o1 + d, 16)])
                              + (t_v[pl.ds(o2 + d, 16)]
                                 + t_v[pl.ds(o3 + d, 16)])
                              + role_v[pl.ds(ro + d, 16)])
                        x.append(xj)
                    s = (x[0] + x[1]) + (x[2] + x[3])
                    q = (x[0] * x[0] + x[1] * x[1]) \
                        + (x[2] * x[2] + x[3] * x[3])
                    mean = xsum(s) * (1.0 / _H)
                    var = xsum(q) * (1.0 / _H) - mean * mean + 1e-5
                    rstd = _rsqrt16(var)
                    ys.append([(x[j] - mean) * rstd * g[j] + be[j]
                               for j in range(4)])
                for k in range(kb, kb + 8):
                    ob = (gi * 16 + k) * _H
                    for j in range(4):
                        out_v[pl.ds(ob + 16 * j, 16)] = ys[k - kb][j]

        n0 = n0 + (_C % _N)
        return jnp.where(n0 >= _N, n0 - _N, n0)

    in_copy(0, sv0, si0).start()
    in_copy(1, sv1, si1).start()

    def cpair(cc, n0):
        for par, sv_v, out_v, si, so in ((0, sv0, ob0, si0, so0),
                                         (1, sv1, ob1, si1, so1)):
            c = cc * 2 + par
            in_copy(c, sv_v, si).wait()

            @pl.when(c >= 2)
            def _():
                out_copy(c - 2, out_v, so).wait()

            n0 = inner(sv_v, out_v, n0)
            out_copy(c, out_v, so).start()

            @pl.when(c + 2 < _NCHUNK)
            def _():
                in_copy(c + 2, sv_v, si).start()
        return n0

    lax.fori_loop(0, _NCHUNK // 2, cpair, 0)
    out_copy(_NCHUNK - 2, ob0, so0).wait()
    out_copy(_NCHUNK - 1, ob1, so1).wait()


_sc_call = pl.kernel(
    _sc_body,
    out_type=jax.ShapeDtypeStruct((_TOK * _H,), jnp.float32),
    mesh=plsc.VectorSubcoreMesh(core_axis_name="c", subcore_axis_name="s"),
    compiler_params=pltpu.CompilerParams(needs_layout_passes=False),
    scratch_types=[
        pltpu.VMEM((_TBL,), jnp.float32),
        pltpu.VMEM((_N * _H,), jnp.float32),
        pltpu.VMEM((_H,), jnp.float32),
        pltpu.VMEM((_H,), jnp.float32),
        pltpu.VMEM((_C + 16,), jnp.int32),
        pltpu.VMEM((_C + 16,), jnp.int32),
        pltpu.VMEM((_C * _H,), jnp.float32),
        pltpu.VMEM((_C * _H,), jnp.float32),
        pltpu.SemaphoreType.DMA,
        pltpu.SemaphoreType.DMA,
        pltpu.SemaphoreType.DMA,
        pltpu.SemaphoreType.DMA,
    ],
)


@jax.jit
def kernel(state_vals, role_table, byte0, byte1, byte2, byte3, W, b,
           gamma, beta):
    t_tbl, role_b = _prep(byte0, byte1, byte2, byte3, W,
                          b.reshape(1, _H), role_table)
    y = _sc_call(state_vals.reshape(_TOK), t_tbl.reshape(_TBL),
                 role_b.reshape(_N * _H), gamma, beta)
    return y.reshape(_B, _N, _H)
